# baseline (speedup 1.0000x reference)
import functools

import jax
import jax.numpy as jnp
from jax import lax
from jax.experimental import pallas as pl
from jax.experimental.pallas import tpu as pltpu
from jax.experimental.pallas import tpu_sc as plsc

B = 4
N = 16384
NQ = 1024
NS1 = 16
NS2 = 32
R1SQ = 0.25
R2SQ = 1.0
NCHUNK = N // 16
_GRP = 8


def _fps_body(x_ref, y_ref, z_ref, nxc_ref, dists_ref):
    X = x_ref[...]
    Y = y_ref[...]
    Z = z_ref[...]
    n_idx = (lax.broadcasted_iota(jnp.int32, (B, 128, 128), 1) * 128
             + lax.broadcasted_iota(jnp.int32, (B, 128, 128), 2))
    dists_ref[...] = jnp.full((B, 128, 128), 1e10, jnp.float32)
    ones11 = jnp.ones((1, 1), jnp.float32)

    def body(i, fa):
        onehot = n_idx == fa
        cx = jnp.sum(jnp.where(onehot, X, 0.0), axis=(1, 2), keepdims=True)
        cy = jnp.sum(jnp.where(onehot, Y, 0.0), axis=(1, 2), keepdims=True)
        cz = jnp.sum(jnp.where(onehot, Z, 0.0), axis=(1, 2), keepdims=True)
        cat = jnp.concatenate(
            [cx.reshape(B, 1), cy.reshape(B, 1), cz.reshape(B, 1)], axis=0)
        row = lax.dot_general(ones11, cat, (((1,), (1,)), ((), ())),
                              preferred_element_type=jnp.float32,
                              precision=lax.Precision.HIGHEST)
        nxc_ref[pl.ds(i, 1), :] = row
        dx = X - cx
        dy = Y - cy
        dz = Z - cz
        d2 = dx * dx + dy * dy
        d2 = d2 + dz * dz
        dnew = jnp.minimum(dists_ref[...], d2)
        dists_ref[...] = dnew
        i2 = jnp.argmax(dnew, axis=2)
        m1v = jnp.max(dnew, axis=2)
        jj = jnp.argmax(m1v, axis=1).reshape(B, 1)
        k_iota = lax.broadcasted_iota(jnp.int32, (B, 128), 1)
        i2sel = jnp.sum(jnp.where(k_iota == jj, i2, 0), axis=1,
                        keepdims=True)
        return (jj * 128 + i2sel).reshape(B, 1, 1)

    lax.fori_loop(0, NQ, body, jnp.zeros((B, 1, 1), jnp.int32))


def _fps(x, y, z):
    return pl.pallas_call(
        _fps_body,
        out_shape=jax.ShapeDtypeStruct((NQ, 3 * B), jnp.float32),
        scratch_shapes=[pltpu.VMEM((B, 128, 128), jnp.float32)],
    )(x, y, z)


_MARGIN = 0.01


def _gmin_body(qx, qy, qz, x_ref, y_ref, z_ref, out_ref, mins):
    ngrp = NCHUNK // _GRP
    bsel = (lax.broadcasted_iota(jnp.int32, (1, B), 1)
            == pl.program_id(0))

    def col(ref):
        return jnp.sum(jnp.where(bsel, ref[0], 0.0), axis=1, keepdims=True)

    qxv = col(qx)
    qyv = col(qy)
    qzv = col(qz)
    for c in range(ngrp):
        xc = x_ref[0, pl.ds(c, 1), :]
        yc = y_ref[0, pl.ds(c, 1), :]
        zc = z_ref[0, pl.ds(c, 1), :]
        dx = xc - qxv
        dy = yc - qyv
        dz = zc - qzv
        d2 = dx * dx + dy * dy
        d2 = d2 + dz * dz
        mins[:, pl.ds(c, 1)] = jnp.min(d2, axis=1, keepdims=True)
    cand = mins[...] <= R2SQ + _MARGIN
    lane = lax.broadcasted_iota(jnp.int32, (128, ngrp), 1)
    s = jnp.where(cand, lane, ngrp)
    sh = 1
    while sh < ngrp:
        shifted = jnp.concatenate(
            [s[:, sh:], jnp.full((128, sh), ngrp, jnp.int32)], axis=1)
        s = jnp.minimum(s, shifted)
        sh *= 2
    out_ref[...] = s


def _gmin(nxc, x3, y3, z3):
    ngrp = NCHUNK // _GRP
    qcols = [nxc[:, 0:B].reshape(NQ // 128, 128, B),
             nxc[:, B:2 * B].reshape(NQ // 128, 128, B),
             nxc[:, 2 * B:3 * B].reshape(NQ // 128, 128, B)]
    return pl.pallas_call(
        _gmin_body,
        grid=(B, NQ // 128),
        in_specs=[
            pl.BlockSpec((1, 128, B), lambda b, qb: (qb, 0, 0)),
            pl.BlockSpec((1, 128, B), lambda b, qb: (qb, 0, 0)),
            pl.BlockSpec((1, 128, B), lambda b, qb: (qb, 0, 0)),
            pl.BlockSpec((1, 128, 128), lambda b, qb: (b, 0, 0)),
            pl.BlockSpec((1, 128, 128), lambda b, qb: (b, 0, 0)),
            pl.BlockSpec((1, 128, 128), lambda b, qb: (b, 0, 0)),
        ],
        out_specs=pl.BlockSpec((128, ngrp), lambda b, qb: (b * (NQ // 128) + qb, 0)),
        out_shape=jax.ShapeDtypeStruct((B * NQ, ngrp), jnp.int32),
        scratch_shapes=[pltpu.VMEM((128, ngrp), jnp.float32)],
    )(*qcols, x3, y3, z3)


_QG = 16


@functools.cache
def _make_sc_group():
    mesh = plsc.VectorSubcoreMesh(core_axis_name="c", subcore_axis_name="s")
    return pl.kernel(
        _sc_group_body,
        mesh=mesh,
        compiler_params=pltpu.CompilerParams(needs_layout_passes=False),
        out_type=[
            jax.ShapeDtypeStruct((B * NQ * NS1 * 8,), jnp.float32),
            jax.ShapeDtypeStruct((B * NQ * NS2 * 8,), jnp.float32),
        ],
        scratch_types=[
            pltpu.VMEM((128 * (NCHUNK // _GRP) + 16,), jnp.int32),
            pltpu.VMEM((N,), jnp.float32),
            pltpu.VMEM((N,), jnp.float32),
            pltpu.VMEM((N,), jnp.float32),
            pltpu.VMEM((N,), jnp.float32),
            pltpu.VMEM((N,), jnp.float32),
            pltpu.VMEM((N,), jnp.float32),
            pltpu.VMEM((128 * 12,), jnp.float32),
            pltpu.VMEM((NS1,), jnp.int32),
            pltpu.VMEM((NS2,), jnp.int32),
            pltpu.VMEM((_QG * NS1 * 8,), jnp.float32),
            pltpu.VMEM((_QG * NS2 * 8,), jnp.float32),
        ],
    )


def _sc_group_body(xp, yp, zp, f0p, f1p, f2p, nxc, gm, out1, out2,
                   gbuf, px, py, pz, f0, f1, f2, nxv, sel1, sel2, st1, st2):
    cid = lax.axis_index("c")
    sid = lax.axis_index("s")
    wid = sid * 2 + cid
    b = wid // 8
    q0 = (wid % 8) * 128

    pltpu.sync_copy(xp.at[b], px)
    pltpu.sync_copy(yp.at[b], py)
    pltpu.sync_copy(zp.at[b], pz)
    pltpu.sync_copy(f0p.at[b], f0)
    pltpu.sync_copy(f1p.at[b], f1)
    pltpu.sync_copy(f2p.at[b], f2)
    pltpu.sync_copy(nxc.at[pl.ds(q0 * 12, 128 * 12)], nxv)
    ngrp = NCHUNK // _GRP
    pltpu.sync_copy(gm.at[pl.ds((b * NQ + q0) * ngrp, 128 * ngrp)],
                    gbuf.at[pl.ds(0, 128 * ngrp)])

    iota16 = lax.iota(jnp.int32, 16)
    zero16i = jnp.zeros((16,), jnp.int32)
    zero16f = jnp.zeros((16,), jnp.float32)

    for k in range(_QG * NS1 * 8 // 16):
        st1[pl.ds(16 * k, 16)] = zero16f
    for k in range(_QG * NS2 * 8 // 16):
        st2[pl.ds(16 * k, 16)] = zero16f

    def qbody(g, qq, _):
        q = g * _QG + qq
        base_c = q * 12 + b
        cxv = plsc.load_gather(nxv, [zero16i + base_c])
        cyv = plsc.load_gather(nxv, [zero16i + (base_c + B)])
        czv = plsc.load_gather(nxv, [zero16i + (base_c + 2 * B)])

        ngrp = NCHUNK // _GRP
        grow = q * ngrp

        def cond(st):
            g, n1, n2, _, _ = st
            return (g < ngrp) & ((n1 < NS1) | (n2 < NS2))

        def scan(st):
            g, n1, n2, f1h, f2h = st
            base0 = g * (16 * _GRP)
            for k in range(_GRP):
                base = base0 + 16 * k
                dx = px[pl.ds(base, 16)] - cxv
                dy = py[pl.ds(base, 16)] - cyv
                dz = pz[pl.ds(base, 16)] - czv
                d2 = dx * dx + dy * dy
                d2 = d2 + dz * dz
                ivec = iota16 + base
                m1 = d2 <= R1SQ
                m2 = d2 <= R2SQ
                cs1 = plsc.cumsum(m1.astype(jnp.int32))
                slots1 = n1 + cs1 - 1
                plsc.store_scatter(sel1, [slots1], ivec,
                                   mask=m1 & (slots1 < NS1))
                cs2 = plsc.cumsum(m2.astype(jnp.int32))
                slots2 = n2 + cs2 - 1
                plsc.store_scatter(sel2, [slots2], ivec,
                                   mask=m2 & (slots2 < NS2))
                cnt1 = jnp.sum(m1.astype(jnp.int32))
                cnt2 = jnp.sum(m2.astype(jnp.int32))
                hit1 = jnp.min(jnp.where(m1, ivec, N))
                hit2 = jnp.min(jnp.where(m2, ivec, N))
                f1h = jnp.where((n1 == 0) & (cnt1 > 0), hit1, f1h)
                f2h = jnp.where((n2 == 0) & (cnt2 > 0), hit2, f2h)
                n1 = n1 + cnt1
                n2 = n2 + cnt2
            gn = gbuf[pl.ds(grow + g + 1, 16)][0]
            gn = jnp.where(g < ngrp - 1, gn, ngrp)
            return (gn, n1, n2, f1h, f2h)

        g0 = gbuf[pl.ds(grow, 16)][0]
        _, n1, n2, f1h, f2h = lax.while_loop(cond, scan, (g0, 0, 0, 0, 0))

        plsc.store_scatter(sel1, [iota16], zero16i + f1h,
                           mask=iota16 >= n1)
        plsc.store_scatter(sel2, [iota16], zero16i + f2h,
                           mask=iota16 >= n2)
        plsc.store_scatter(sel2, [iota16 + 16], zero16i + f2h,
                           mask=iota16 + 16 >= n2)

        for sel, nsamp, stf in ((sel1, NS1, st1), (sel2, NS2, st2)):
            qoff = qq * nsamp * 8
            for k in range(nsamp // 16):
                idxv = sel[pl.ds(16 * k, 16)]
                rows = qoff + (iota16 + 16 * k) * 8
                for c, tab in enumerate((px, py, pz, f0, f1, f2)):
                    vals = plsc.load_gather(tab, [idxv])
                    if c == 0:
                        vals = vals - cxv
                    elif c == 1:
                        vals = vals - cyv
                    elif c == 2:
                        vals = vals - czv
                    plsc.store_scatter(stf, [rows + c], vals)
        return 0

    def gbody(g, _):
        lax.fori_loop(0, _QG, functools.partial(qbody, g), 0)
        base = (b * NQ + q0 + g * _QG)
        pltpu.sync_copy(st1, out1.at[pl.ds(base * NS1 * 8, _QG * NS1 * 8)])
        pltpu.sync_copy(st2, out2.at[pl.ds(base * NS2 * 8, _QG * NS2 * 8)])
        return 0

    lax.fori_loop(0, 128 // _QG, gbody, 0)


def _mlp_body(g1_ref, g2_ref, w10, b10, w11, b11, w12, b12,
              w20, b20, w21, b21, w22, b22, out_ref):
    def mlp(g, w0, b0, w1, b1, w2, b2, ns):
        h = jnp.maximum(jnp.dot(g, w0[...],
                                preferred_element_type=jnp.float32)
                        + b0[...], 0.0)
        h = jnp.maximum(jnp.dot(h, w1[...],
                                preferred_element_type=jnp.float32)
                        + b1[...], 0.0)
        h = jnp.maximum(jnp.dot(h, w2[...],
                                preferred_element_type=jnp.float32)
                        + b2[...], 0.0)
        return jnp.max(h.reshape(128, ns, 128), axis=1)

    o1 = mlp(g1_ref[...], w10, b10, w11, b11, w12, b12, NS1)
    o2 = mlp(g2_ref[...], w20, b20, w21, b21, w22, b22, NS2)
    out_ref[...] = jnp.concatenate([o1, o2], axis=1)


def _mlp(g1, g2, ws):
    nblk = B * NQ // 128
    wspecs = [pl.BlockSpec(w.shape, lambda i: (0, 0)) for w in ws]
    return pl.pallas_call(
        _mlp_body,
        grid=(nblk,),
        in_specs=[
            pl.BlockSpec((128 * NS1, 8), lambda i: (i, 0)),
            pl.BlockSpec((128 * NS2, 8), lambda i: (i, 0)),
            *wspecs,
        ],
        out_specs=pl.BlockSpec((128, 256), lambda i: (i, 0)),
        out_shape=jax.ShapeDtypeStruct((B * NQ, 256), jnp.float32),
    )(g1, g2, *ws)


def kernel(pointcloud, w1_0, b1_0, w1_1, b1_1, w1_2, b1_2,
           w2_0, b2_0, w2_1, b2_1, w2_2, b2_2):
    x = pointcloud[:, :, 0]
    y = pointcloud[:, :, 1]
    z = pointcloud[:, :, 2]
    f0 = pointcloud[:, :, 3]
    f1 = pointcloud[:, :, 4]
    f2 = pointcloud[:, :, 5]

    x3 = x.reshape(B, 128, 128)
    y3 = y.reshape(B, 128, 128)
    z3 = z.reshape(B, 128, 128)
    nxc = _fps(x3, y3, z3)
    gmn = _gmin(nxc, x3, y3, z3)

    g1, g2 = _make_sc_group()(x, y, z, f0, f1, f2, nxc.reshape(NQ * 3 * B),
                              gmn.reshape(B * NQ * (NCHUNK // _GRP)))
    g1 = g1.reshape(B * NQ * NS1, 8)
    g2 = g2.reshape(B * NQ * NS2, 8)

    def pad8(w):
        return jnp.zeros((8, w.shape[1]), jnp.float32).at[:6].set(w)

    ws = [pad8(w1_0), b1_0.reshape(1, -1), w1_1, b1_1.reshape(1, -1),
          w1_2, b1_2.reshape(1, -1),
          pad8(w2_0), b2_0.reshape(1, -1), w2_1, b2_1.reshape(1, -1),
          w2_2, b2_2.reshape(1, -1)]
    out = _mlp(g1, g2, ws)
    return out.reshape(B, NQ, 256)


# speedup vs baseline: 1.2434x; 1.2434x over previous
import functools

import jax
import jax.numpy as jnp
from jax import lax
from jax.experimental import pallas as pl
from jax.experimental.pallas import tpu as pltpu
from jax.experimental.pallas import tpu_sc as plsc

B = 4
N = 16384
NQ = 1024
NS1 = 16
NS2 = 32
R1SQ = 0.25
R2SQ = 1.0
NCHUNK = N // 16
_GRP = 8


def _fps_body(x_ref, y_ref, z_ref, nxc_ref, dists_ref):
    X = x_ref[...]
    Y = y_ref[...]
    Z = z_ref[...]
    n_idx = (lax.broadcasted_iota(jnp.int32, (B, 128, 128), 1) * 128
             + lax.broadcasted_iota(jnp.int32, (B, 128, 128), 2))
    dists_ref[...] = jnp.full((B, 128, 128), 1e10, jnp.float32)
    ones11 = jnp.ones((1, 1), jnp.float32)

    def body(i, fa):
        onehot = n_idx == fa
        cx = jnp.sum(jnp.where(onehot, X, 0.0), axis=(1, 2), keepdims=True)
        cy = jnp.sum(jnp.where(onehot, Y, 0.0), axis=(1, 2), keepdims=True)
        cz = jnp.sum(jnp.where(onehot, Z, 0.0), axis=(1, 2), keepdims=True)
        cat = jnp.concatenate(
            [cx.reshape(B, 1), cy.reshape(B, 1), cz.reshape(B, 1)], axis=0)
        row = lax.dot_general(ones11, cat, (((1,), (1,)), ((), ())),
                              preferred_element_type=jnp.float32,
                              precision=lax.Precision.HIGHEST)
        nxc_ref[pl.ds(i, 1), :] = row
        dx = X - cx
        dy = Y - cy
        dz = Z - cz
        d2 = dx * dx + dy * dy
        d2 = d2 + dz * dz
        dnew = jnp.minimum(dists_ref[...], d2)
        dists_ref[...] = dnew
        i2 = jnp.argmax(dnew, axis=2)
        m1v = jnp.max(dnew, axis=2)
        jj = jnp.argmax(m1v, axis=1).reshape(B, 1)
        k_iota = lax.broadcasted_iota(jnp.int32, (B, 128), 1)
        i2sel = jnp.sum(jnp.where(k_iota == jj, i2, 0), axis=1,
                        keepdims=True)
        return (jj * 128 + i2sel).reshape(B, 1, 1)

    lax.fori_loop(0, NQ, body, jnp.zeros((B, 1, 1), jnp.int32))


def _fps(x, y, z):
    return pl.pallas_call(
        _fps_body,
        out_shape=jax.ShapeDtypeStruct((NQ, 3 * B), jnp.float32),
        scratch_shapes=[pltpu.VMEM((B, 128, 128), jnp.float32)],
    )(x, y, z)


_QG = 16


@functools.cache
def _make_sc_group():
    mesh = plsc.VectorSubcoreMesh(core_axis_name="c", subcore_axis_name="s")
    return pl.kernel(
        _sc_group_body,
        mesh=mesh,
        compiler_params=pltpu.CompilerParams(needs_layout_passes=False),
        out_type=[
            jax.ShapeDtypeStruct((B * NQ * NS1 * 8,), jnp.float32),
            jax.ShapeDtypeStruct((B * NQ * NS2 * 8,), jnp.float32),
        ],
        scratch_types=[
            pltpu.VMEM((N,), jnp.float32),
            pltpu.VMEM((N,), jnp.float32),
            pltpu.VMEM((N,), jnp.float32),
            pltpu.VMEM((N,), jnp.float32),
            pltpu.VMEM((N,), jnp.float32),
            pltpu.VMEM((N,), jnp.float32),
            pltpu.VMEM((128 * 12,), jnp.float32),
            pltpu.VMEM((NS1,), jnp.int32),
            pltpu.VMEM((NS2,), jnp.int32),
            pltpu.VMEM((_QG * NS1 * 8,), jnp.float32),
            pltpu.VMEM((_QG * NS2 * 8,), jnp.float32),
        ],
    )


def _sc_group_body(xp, yp, zp, f0p, f1p, f2p, nxc, out1, out2,
                   px, py, pz, f0, f1, f2, nxv, sel1, sel2, st1, st2):
    cid = lax.axis_index("c")
    sid = lax.axis_index("s")
    wid = sid * 2 + cid
    b = wid // 8
    q0 = (wid % 8) * 128

    pltpu.sync_copy(xp.at[b], px)
    pltpu.sync_copy(yp.at[b], py)
    pltpu.sync_copy(zp.at[b], pz)
    pltpu.sync_copy(f0p.at[b], f0)
    pltpu.sync_copy(f1p.at[b], f1)
    pltpu.sync_copy(f2p.at[b], f2)
    pltpu.sync_copy(nxc.at[pl.ds(q0 * 12, 128 * 12)], nxv)

    iota16 = lax.iota(jnp.int32, 16)
    zero16i = jnp.zeros((16,), jnp.int32)
    zero16f = jnp.zeros((16,), jnp.float32)

    for k in range(_QG * NS1 * 8 // 16):
        st1[pl.ds(16 * k, 16)] = zero16f
    for k in range(_QG * NS2 * 8 // 16):
        st2[pl.ds(16 * k, 16)] = zero16f

    def qbody(g, qq, _):
        q = g * _QG + qq
        base_c = q * 12 + b
        cxv = plsc.load_gather(nxv, [zero16i + base_c])
        cyv = plsc.load_gather(nxv, [zero16i + (base_c + B)])
        czv = plsc.load_gather(nxv, [zero16i + (base_c + 2 * B)])

        def cond(st):
            g, n1, n2, _, _ = st
            return (g < NCHUNK // _GRP) & ((n1 < NS1) | (n2 < NS2))

        def scan(st):
            g, n1, n2, f1h, f2h = st
            base0 = g * (16 * _GRP)
            d2s = []
            m2s = []
            for k in range(_GRP):
                base = base0 + 16 * k
                dx = px[pl.ds(base, 16)] - cxv
                dy = py[pl.ds(base, 16)] - cyv
                dz = pz[pl.ds(base, 16)] - czv
                d2 = dx * dx + dy * dy
                d2 = d2 + dz * dz
                d2s.append(d2)
                m2s.append(d2 <= R2SQ)
            anyv = m2s[0]
            for k in range(1, _GRP):
                anyv = anyv | m2s[k]

            def compact(carry):
                n1, n2, f1h, f2h = carry
                for k in range(_GRP):
                    ivec = iota16 + (base0 + 16 * k)
                    d2 = d2s[k]
                    m1 = d2 <= R1SQ
                    m2 = m2s[k]
                    cs1 = plsc.cumsum(m1.astype(jnp.int32))
                    slots1 = n1 + cs1 - 1
                    plsc.store_scatter(sel1, [slots1], ivec,
                                       mask=m1 & (slots1 < NS1))
                    cs2 = plsc.cumsum(m2.astype(jnp.int32))
                    slots2 = n2 + cs2 - 1
                    plsc.store_scatter(sel2, [slots2], ivec,
                                       mask=m2 & (slots2 < NS2))
                    cnt1 = jnp.sum(m1.astype(jnp.int32))
                    cnt2 = jnp.sum(m2.astype(jnp.int32))
                    hit1 = jnp.min(jnp.where(m1, ivec, N))
                    hit2 = jnp.min(jnp.where(m2, ivec, N))
                    f1h = jnp.where((n1 == 0) & (cnt1 > 0), hit1, f1h)
                    f2h = jnp.where((n2 == 0) & (cnt2 > 0), hit2, f2h)
                    n1 = n1 + cnt1
                    n2 = n2 + cnt2
                return (n1, n2, f1h, f2h)

            n1, n2, f1h, f2h = lax.cond(jnp.any(anyv), compact,
                                        lambda c: c, (n1, n2, f1h, f2h))
            return (g + 1, n1, n2, f1h, f2h)

        _, n1, n2, f1h, f2h = lax.while_loop(cond, scan, (0, 0, 0, 0, 0))

        plsc.store_scatter(sel1, [iota16], zero16i + f1h,
                           mask=iota16 >= n1)
        plsc.store_scatter(sel2, [iota16], zero16i + f2h,
                           mask=iota16 >= n2)
        plsc.store_scatter(sel2, [iota16 + 16], zero16i + f2h,
                           mask=iota16 + 16 >= n2)

        for sel, nsamp, stf in ((sel1, NS1, st1), (sel2, NS2, st2)):
            qoff = qq * nsamp * 8
            for k in range(nsamp // 16):
                idxv = sel[pl.ds(16 * k, 16)]
                rows = qoff + (iota16 + 16 * k) * 8
                for c, tab in enumerate((px, py, pz, f0, f1, f2)):
                    vals = plsc.load_gather(tab, [idxv])
                    if c == 0:
                        vals = vals - cxv
                    elif c == 1:
                        vals = vals - cyv
                    elif c == 2:
                        vals = vals - czv
                    plsc.store_scatter(stf, [rows + c], vals)
        return 0

    def gbody(g, _):
        lax.fori_loop(0, _QG, functools.partial(qbody, g), 0)
        base = (b * NQ + q0 + g * _QG)
        pltpu.sync_copy(st1, out1.at[pl.ds(base * NS1 * 8, _QG * NS1 * 8)])
        pltpu.sync_copy(st2, out2.at[pl.ds(base * NS2 * 8, _QG * NS2 * 8)])
        return 0

    lax.fori_loop(0, 128 // _QG, gbody, 0)


def _mlp_body(g1_ref, g2_ref, w10, b10, w11, b11, w12, b12,
              w20, b20, w21, b21, w22, b22, out_ref):
    def mlp(g, w0, b0, w1, b1, w2, b2, ns):
        h = jnp.maximum(jnp.dot(g, w0[...],
                                preferred_element_type=jnp.float32)
                        + b0[...], 0.0)
        h = jnp.maximum(jnp.dot(h, w1[...],
                                preferred_element_type=jnp.float32)
                        + b1[...], 0.0)
        h = jnp.maximum(jnp.dot(h, w2[...],
                                preferred_element_type=jnp.float32)
                        + b2[...], 0.0)
        return jnp.max(h.reshape(128, ns, 128), axis=1)

    o1 = mlp(g1_ref[...], w10, b10, w11, b11, w12, b12, NS1)
    o2 = mlp(g2_ref[...], w20, b20, w21, b21, w22, b22, NS2)
    out_ref[...] = jnp.concatenate([o1, o2], axis=1)


def _mlp(g1, g2, ws):
    nblk = B * NQ // 128
    wspecs = [pl.BlockSpec(w.shape, lambda i: (0, 0)) for w in ws]
    return pl.pallas_call(
        _mlp_body,
        grid=(nblk,),
        in_specs=[
            pl.BlockSpec((128 * NS1, 8), lambda i: (i, 0)),
            pl.BlockSpec((128 * NS2, 8), lambda i: (i, 0)),
            *wspecs,
        ],
        out_specs=pl.BlockSpec((128, 256), lambda i: (i, 0)),
        out_shape=jax.ShapeDtypeStruct((B * NQ, 256), jnp.float32),
    )(g1, g2, *ws)


def kernel(pointcloud, w1_0, b1_0, w1_1, b1_1, w1_2, b1_2,
           w2_0, b2_0, w2_1, b2_1, w2_2, b2_2):
    x = pointcloud[:, :, 0]
    y = pointcloud[:, :, 1]
    z = pointcloud[:, :, 2]
    f0 = pointcloud[:, :, 3]
    f1 = pointcloud[:, :, 4]
    f2 = pointcloud[:, :, 5]

    nxc = _fps(x.reshape(B, 128, 128), y.reshape(B, 128, 128),
               z.reshape(B, 128, 128))

    g1, g2 = _make_sc_group()(x, y, z, f0, f1, f2, nxc.reshape(NQ * 3 * B))
    g1 = g1.reshape(B * NQ * NS1, 8)
    g2 = g2.reshape(B * NQ * NS2, 8)

    def pad8(w):
        return jnp.zeros((8, w.shape[1]), jnp.float32).at[:6].set(w)

    ws = [pad8(w1_0), b1_0.reshape(1, -1), w1_1, b1_1.reshape(1, -1),
          w1_2, b1_2.reshape(1, -1),
          pad8(w2_0), b2_0.reshape(1, -1), w2_1, b2_1.reshape(1, -1),
          w2_2, b2_2.reshape(1, -1)]
    out = _mlp(g1, g2, ws)
    return out.reshape(B, NQ, 256)


# speedup vs baseline: 1.2441x; 1.0005x over previous
import functools

import jax
import jax.numpy as jnp
from jax import lax
from jax.experimental import pallas as pl
from jax.experimental.pallas import tpu as pltpu
from jax.experimental.pallas import tpu_sc as plsc

B = 4
N = 16384
NQ = 1024
NS1 = 16
NS2 = 32
R1SQ = 0.25
R2SQ = 1.0
NCHUNK = N // 16
_GRP = 8


def _fps_body(x_ref, y_ref, z_ref, nxc_ref, dists_ref):
    X = x_ref[...]
    Y = y_ref[...]
    Z = z_ref[...]
    n_idx = (lax.broadcasted_iota(jnp.int32, (B, 128, 128), 1) * 128
             + lax.broadcasted_iota(jnp.int32, (B, 128, 128), 2))
    dists_ref[...] = jnp.full((B, 128, 128), 1e10, jnp.float32)
    ones11 = jnp.ones((1, 1), jnp.float32)

    def body(i, fa):
        onehot = n_idx == fa
        cx = jnp.sum(jnp.where(onehot, X, 0.0), axis=(1, 2), keepdims=True)
        cy = jnp.sum(jnp.where(onehot, Y, 0.0), axis=(1, 2), keepdims=True)
        cz = jnp.sum(jnp.where(onehot, Z, 0.0), axis=(1, 2), keepdims=True)
        cat = jnp.concatenate(
            [cx.reshape(B, 1), cy.reshape(B, 1), cz.reshape(B, 1)], axis=0)
        row = lax.dot_general(ones11, cat, (((1,), (1,)), ((), ())),
                              preferred_element_type=jnp.float32,
                              precision=lax.Precision.HIGHEST)
        nxc_ref[pl.ds(i, 1), :] = row
        dx = X - cx
        dy = Y - cy
        dz = Z - cz
        d2 = dx * dx + dy * dy
        d2 = d2 + dz * dz
        dnew = jnp.minimum(dists_ref[...], d2)
        dists_ref[...] = dnew
        i2 = jnp.argmax(dnew, axis=2)
        m1v = jnp.max(dnew, axis=2)
        jj = jnp.argmax(m1v, axis=1).reshape(B, 1)
        k_iota = lax.broadcasted_iota(jnp.int32, (B, 128), 1)
        i2sel = jnp.sum(jnp.where(k_iota == jj, i2, 0), axis=1,
                        keepdims=True)
        return (jj * 128 + i2sel).reshape(B, 1, 1)

    lax.fori_loop(0, NQ, body, jnp.zeros((B, 1, 1), jnp.int32))


def _fps(x, y, z):
    return pl.pallas_call(
        _fps_body,
        out_shape=jax.ShapeDtypeStruct((NQ, 3 * B), jnp.float32),
        scratch_shapes=[pltpu.VMEM((B, 128, 128), jnp.float32)],
    )(x, y, z)


_QG = 16


@functools.cache
def _make_sc_group():
    mesh = plsc.VectorSubcoreMesh(core_axis_name="c", subcore_axis_name="s")
    return pl.kernel(
        _sc_group_body,
        mesh=mesh,
        compiler_params=pltpu.CompilerParams(needs_layout_passes=False),
        out_type=[
            jax.ShapeDtypeStruct((B * NQ * NS1 * 8,), jnp.float32),
            jax.ShapeDtypeStruct((B * NQ * NS2 * 8,), jnp.float32),
        ],
        scratch_types=[
            pltpu.VMEM((N,), jnp.float32),
            pltpu.VMEM((N,), jnp.float32),
            pltpu.VMEM((N,), jnp.float32),
            pltpu.VMEM((N,), jnp.float32),
            pltpu.VMEM((N,), jnp.float32),
            pltpu.VMEM((N,), jnp.float32),
            pltpu.VMEM((128 * 12,), jnp.float32),
            pltpu.VMEM((NS1,), jnp.int32),
            pltpu.VMEM((NS2,), jnp.int32),
            pltpu.VMEM((_QG * NS1 * 8,), jnp.float32),
            pltpu.VMEM((_QG * NS2 * 8,), jnp.float32),
        ],
    )


def _sc_group_body(xp, yp, zp, f0p, f1p, f2p, nxc, out1, out2,
                   px, py, pz, f0, f1, f2, nxv, sel1, sel2, st1, st2):
    cid = lax.axis_index("c")
    sid = lax.axis_index("s")
    wid = sid * 2 + cid
    b = wid // 8
    q0 = (wid % 8) * 128

    pltpu.sync_copy(xp.at[b], px)
    pltpu.sync_copy(yp.at[b], py)
    pltpu.sync_copy(zp.at[b], pz)
    pltpu.sync_copy(f0p.at[b], f0)
    pltpu.sync_copy(f1p.at[b], f1)
    pltpu.sync_copy(f2p.at[b], f2)
    pltpu.sync_copy(nxc.at[pl.ds(q0 * 12, 128 * 12)], nxv)

    iota16 = lax.iota(jnp.int32, 16)
    zero16i = jnp.zeros((16,), jnp.int32)
    zero16f = jnp.zeros((16,), jnp.float32)

    for k in range(_QG * NS1 * 8 // 16):
        st1[pl.ds(16 * k, 16)] = zero16f
    for k in range(_QG * NS2 * 8 // 16):
        st2[pl.ds(16 * k, 16)] = zero16f

    def qbody(g, qq, _):
        q = g * _QG + qq
        base_c = q * 12 + b
        cxv = plsc.load_gather(nxv, [zero16i + base_c])
        cyv = plsc.load_gather(nxv, [zero16i + (base_c + B)])
        czv = plsc.load_gather(nxv, [zero16i + (base_c + 2 * B)])

        def cond(st):
            g, n1, n2, _, _ = st
            return (g < NCHUNK // _GRP) & ((n1 < NS1) | (n2 < NS2))

        def scan(st):
            g, n1, n2, f1h, f2h = st
            base0 = g * (16 * _GRP)
            d2s = []
            m2s = []
            for k in range(_GRP):
                base = base0 + 16 * k
                dx = px[pl.ds(base, 16)] - cxv
                dy = py[pl.ds(base, 16)] - cyv
                dz = pz[pl.ds(base, 16)] - czv
                d2 = dx * dx + dy * dy
                d2 = d2 + dz * dz
                d2s.append(d2)
                m2s.append(d2 <= R2SQ)
            anyv = m2s[0]
            for k in range(1, _GRP):
                anyv = anyv | m2s[k]

            def compact(carry):
                n1, n2, f1h, f2h = carry
                for k in range(_GRP):
                    ivec = iota16 + (base0 + 16 * k)
                    d2 = d2s[k]
                    m1 = d2 <= R1SQ
                    m2 = m2s[k]
                    cs1 = plsc.cumsum(m1.astype(jnp.int32))
                    slots1 = n1 + cs1 - 1
                    plsc.store_scatter(sel1, [slots1], ivec,
                                       mask=m1 & (slots1 < NS1))
                    cs2 = plsc.cumsum(m2.astype(jnp.int32))
                    slots2 = n2 + cs2 - 1
                    plsc.store_scatter(sel2, [slots2], ivec,
                                       mask=m2 & (slots2 < NS2))
                    cnt1 = cs1[15]
                    cnt2 = cs2[15]
                    hit1 = jnp.min(jnp.where(m1, ivec, N))
                    hit2 = jnp.min(jnp.where(m2, ivec, N))
                    f1h = jnp.where((n1 == 0) & (cnt1 > 0), hit1, f1h)
                    f2h = jnp.where((n2 == 0) & (cnt2 > 0), hit2, f2h)
                    n1 = n1 + cnt1
                    n2 = n2 + cnt2
                return (n1, n2, f1h, f2h)

            n1, n2, f1h, f2h = lax.cond(jnp.any(anyv), compact,
                                        lambda c: c, (n1, n2, f1h, f2h))
            return (g + 1, n1, n2, f1h, f2h)

        _, n1, n2, f1h, f2h = lax.while_loop(cond, scan, (0, 0, 0, 0, 0))

        plsc.store_scatter(sel1, [iota16], zero16i + f1h,
                           mask=iota16 >= n1)
        plsc.store_scatter(sel2, [iota16], zero16i + f2h,
                           mask=iota16 >= n2)
        plsc.store_scatter(sel2, [iota16 + 16], zero16i + f2h,
                           mask=iota16 + 16 >= n2)

        for sel, nsamp, stf in ((sel1, NS1, st1), (sel2, NS2, st2)):
            qoff = qq * nsamp * 8
            for k in range(nsamp // 16):
                idxv = sel[pl.ds(16 * k, 16)]
                rows = qoff + (iota16 + 16 * k) * 8
                for c, tab in enumerate((px, py, pz, f0, f1, f2)):
                    vals = plsc.load_gather(tab, [idxv])
                    if c == 0:
                        vals = vals - cxv
                    elif c == 1:
                        vals = vals - cyv
                    elif c == 2:
                        vals = vals - czv
                    plsc.store_scatter(stf, [rows + c], vals)
        return 0

    def gbody(g, _):
        lax.fori_loop(0, _QG, functools.partial(qbody, g), 0)
        base = (b * NQ + q0 + g * _QG)
        pltpu.sync_copy(st1, out1.at[pl.ds(base * NS1 * 8, _QG * NS1 * 8)])
        pltpu.sync_copy(st2, out2.at[pl.ds(base * NS2 * 8, _QG * NS2 * 8)])
        return 0

    lax.fori_loop(0, 128 // _QG, gbody, 0)


def _mlp_body(g1_ref, g2_ref, w10, b10, w11, b11, w12, b12,
              w20, b20, w21, b21, w22, b22, out_ref):
    def mlp(g, w0, b0, w1, b1, w2, b2, ns):
        h = jnp.maximum(jnp.dot(g, w0[...],
                                preferred_element_type=jnp.float32)
                        + b0[...], 0.0)
        h = jnp.maximum(jnp.dot(h, w1[...],
                                preferred_element_type=jnp.float32)
                        + b1[...], 0.0)
        h = jnp.maximum(jnp.dot(h, w2[...],
                                preferred_element_type=jnp.float32)
                        + b2[...], 0.0)
        return jnp.max(h.reshape(128, ns, 128), axis=1)

    o1 = mlp(g1_ref[...], w10, b10, w11, b11, w12, b12, NS1)
    o2 = mlp(g2_ref[...], w20, b20, w21, b21, w22, b22, NS2)
    out_ref[...] = jnp.concatenate([o1, o2], axis=1)


def _mlp(g1, g2, ws):
    nblk = B * NQ // 128
    wspecs = [pl.BlockSpec(w.shape, lambda i: (0, 0)) for w in ws]
    return pl.pallas_call(
        _mlp_body,
        grid=(nblk,),
        in_specs=[
            pl.BlockSpec((128 * NS1, 8), lambda i: (i, 0)),
            pl.BlockSpec((128 * NS2, 8), lambda i: (i, 0)),
            *wspecs,
        ],
        out_specs=pl.BlockSpec((128, 256), lambda i: (i, 0)),
        out_shape=jax.ShapeDtypeStruct((B * NQ, 256), jnp.float32),
    )(g1, g2, *ws)


def kernel(pointcloud, w1_0, b1_0, w1_1, b1_1, w1_2, b1_2,
           w2_0, b2_0, w2_1, b2_1, w2_2, b2_2):
    x = pointcloud[:, :, 0]
    y = pointcloud[:, :, 1]
    z = pointcloud[:, :, 2]
    f0 = pointcloud[:, :, 3]
    f1 = pointcloud[:, :, 4]
    f2 = pointcloud[:, :, 5]

    nxc = _fps(x.reshape(B, 128, 128), y.reshape(B, 128, 128),
               z.reshape(B, 128, 128))

    g1, g2 = _make_sc_group()(x, y, z, f0, f1, f2, nxc.reshape(NQ * 3 * B))
    g1 = g1.reshape(B * NQ * NS1, 8)
    g2 = g2.reshape(B * NQ * NS2, 8)

    def pad8(w):
        return jnp.zeros((8, w.shape[1]), jnp.float32).at[:6].set(w)

    ws = [pad8(w1_0), b1_0.reshape(1, -1), w1_1, b1_1.reshape(1, -1),
          w1_2, b1_2.reshape(1, -1),
          pad8(w2_0), b2_0.reshape(1, -1), w2_1, b2_1.reshape(1, -1),
          w2_2, b2_2.reshape(1, -1)]
    out = _mlp(g1, g2, ws)
    return out.reshape(B, NQ, 256)


# speedup vs baseline: 1.2794x; 1.0284x over previous
import functools

import jax
import jax.numpy as jnp
from jax import lax
from jax.experimental import pallas as pl
from jax.experimental.pallas import tpu as pltpu
from jax.experimental.pallas import tpu_sc as plsc

B = 4
N = 16384
NQ = 1024
NS1 = 16
NS2 = 32
R1SQ = 0.25
R2SQ = 1.0
NCHUNK = N // 16
_GRP = 8


def _fps_body(x_ref, y_ref, z_ref, nxc_ref, dists_ref):
    X = x_ref[...]
    Y = y_ref[...]
    Z = z_ref[...]
    n_idx = (lax.broadcasted_iota(jnp.int32, (B, 128, 128), 1) * 128
             + lax.broadcasted_iota(jnp.int32, (B, 128, 128), 2))
    dists_ref[...] = jnp.full((B, 128, 128), 1e10, jnp.float32)
    ones11 = jnp.ones((1, 1), jnp.float32)

    def body(i, fa):
        onehot = n_idx == fa
        cx = jnp.sum(jnp.where(onehot, X, 0.0), axis=(1, 2), keepdims=True)
        cy = jnp.sum(jnp.where(onehot, Y, 0.0), axis=(1, 2), keepdims=True)
        cz = jnp.sum(jnp.where(onehot, Z, 0.0), axis=(1, 2), keepdims=True)
        cat = jnp.concatenate(
            [cx.reshape(B, 1), cy.reshape(B, 1), cz.reshape(B, 1)], axis=0)
        row = lax.dot_general(ones11, cat, (((1,), (1,)), ((), ())),
                              preferred_element_type=jnp.float32,
                              precision=lax.Precision.HIGHEST)
        nxc_ref[pl.ds(i, 1), :] = row
        dx = X - cx
        dy = Y - cy
        dz = Z - cz
        d2 = dx * dx + dy * dy
        d2 = d2 + dz * dz
        dnew = jnp.minimum(dists_ref[...], d2)
        dists_ref[...] = dnew
        i2 = jnp.argmax(dnew, axis=2)
        m1v = jnp.max(dnew, axis=2)
        jj = jnp.argmax(m1v, axis=1).reshape(B, 1)
        k_iota = lax.broadcasted_iota(jnp.int32, (B, 128), 1)
        i2sel = jnp.sum(jnp.where(k_iota == jj, i2, 0), axis=1,
                        keepdims=True)
        return (jj * 128 + i2sel).reshape(B, 1, 1)

    lax.fori_loop(0, NQ, body, jnp.zeros((B, 1, 1), jnp.int32))


def _fps(x, y, z):
    return pl.pallas_call(
        _fps_body,
        out_shape=jax.ShapeDtypeStruct((NQ, 3 * B), jnp.float32),
        scratch_shapes=[pltpu.VMEM((B, 128, 128), jnp.float32)],
    )(x, y, z)


_QG = 16


@functools.cache
def _make_sc_group():
    mesh = plsc.VectorSubcoreMesh(core_axis_name="c", subcore_axis_name="s")
    return pl.kernel(
        _sc_group_body,
        mesh=mesh,
        compiler_params=pltpu.CompilerParams(needs_layout_passes=False),
        out_type=[
            jax.ShapeDtypeStruct((B * NQ * NS1 * 8,), jnp.float32),
            jax.ShapeDtypeStruct((B * NQ * NS2 * 8,), jnp.float32),
        ],
        scratch_types=[
            pltpu.VMEM((N,), jnp.float32),
            pltpu.VMEM((N,), jnp.float32),
            pltpu.VMEM((N,), jnp.float32),
            pltpu.VMEM((N,), jnp.float32),
            pltpu.VMEM((N,), jnp.float32),
            pltpu.VMEM((N,), jnp.float32),
            pltpu.VMEM((128 * 12,), jnp.float32),
            pltpu.VMEM((NS1,), jnp.int32),
            pltpu.VMEM((NS2,), jnp.int32),
            pltpu.VMEM((_QG * NS1 * 8,), jnp.float32),
            pltpu.VMEM((_QG * NS2 * 8,), jnp.float32),
        ],
    )


def _sc_group_body(xp, yp, zp, f0p, f1p, f2p, nxc, out1, out2,
                   px, py, pz, f0, f1, f2, nxv, sel1, sel2, st1, st2):
    cid = lax.axis_index("c")
    sid = lax.axis_index("s")
    wid = sid * 2 + cid
    b = wid // 8
    q0 = (wid % 8) * 128

    pltpu.sync_copy(xp.at[b], px)
    pltpu.sync_copy(yp.at[b], py)
    pltpu.sync_copy(zp.at[b], pz)
    pltpu.sync_copy(f0p.at[b], f0)
    pltpu.sync_copy(f1p.at[b], f1)
    pltpu.sync_copy(f2p.at[b], f2)
    pltpu.sync_copy(nxc.at[pl.ds(q0 * 12, 128 * 12)], nxv)

    iota16 = lax.iota(jnp.int32, 16)
    zero16i = jnp.zeros((16,), jnp.int32)
    zero16f = jnp.zeros((16,), jnp.float32)

    for k in range(_QG * NS1 * 8 // 16):
        st1[pl.ds(16 * k, 16)] = zero16f
    for k in range(_QG * NS2 * 8 // 16):
        st2[pl.ds(16 * k, 16)] = zero16f

    def qbody(g, qq, _):
        q = g * _QG + qq
        base_c = q * 12 + b
        cxv = plsc.load_gather(nxv, [zero16i + base_c])
        cyv = plsc.load_gather(nxv, [zero16i + (base_c + B)])
        czv = plsc.load_gather(nxv, [zero16i + (base_c + 2 * B)])

        def cond(st):
            g, n1, n2 = st
            return (g < NCHUNK // _GRP) & ((n1 < NS1) | (n2 < NS2))

        def scan(st):
            g, n1, n2 = st
            base0 = g * (16 * _GRP)
            d2s = []
            m2s = []
            for k in range(_GRP):
                base = base0 + 16 * k
                dx = px[pl.ds(base, 16)] - cxv
                dy = py[pl.ds(base, 16)] - cyv
                dz = pz[pl.ds(base, 16)] - czv
                d2 = dx * dx + dy * dy
                d2 = d2 + dz * dz
                d2s.append(d2)
                m2s.append(d2 <= R2SQ)
            anyv = m2s[0]
            for k in range(1, _GRP):
                anyv = anyv | m2s[k]

            def compact(carry):
                n1, n2 = carry
                for k in range(_GRP):
                    ivec = iota16 + (base0 + 16 * k)
                    d2 = d2s[k]
                    m1 = d2 <= R1SQ
                    m2 = m2s[k]
                    cs1 = plsc.cumsum(m1.astype(jnp.int32))
                    slots1 = n1 + cs1 - 1
                    plsc.store_scatter(sel1, [slots1], ivec,
                                       mask=m1 & (slots1 < NS1))
                    cs2 = plsc.cumsum(m2.astype(jnp.int32))
                    slots2 = n2 + cs2 - 1
                    plsc.store_scatter(sel2, [slots2], ivec,
                                       mask=m2 & (slots2 < NS2))
                    n1 = n1 + cs1[15]
                    n2 = n2 + cs2[15]
                return (n1, n2)

            n1, n2 = lax.cond(jnp.any(anyv), compact,
                              lambda c: c, (n1, n2))
            return (g + 1, n1, n2)

        _, n1, n2 = lax.while_loop(cond, scan, (0, 0, 0))

        f1h = jnp.where(n1 > 0, sel1[pl.ds(0, 16)][0], 0)
        f2h = jnp.where(n2 > 0, sel2[pl.ds(0, 16)][0], 0)
        plsc.store_scatter(sel1, [iota16], zero16i + f1h,
                           mask=iota16 >= n1)
        plsc.store_scatter(sel2, [iota16], zero16i + f2h,
                           mask=iota16 >= n2)
        plsc.store_scatter(sel2, [iota16 + 16], zero16i + f2h,
                           mask=iota16 + 16 >= n2)

        for sel, nsamp, stf in ((sel1, NS1, st1), (sel2, NS2, st2)):
            qoff = qq * nsamp * 8
            for k in range(nsamp // 16):
                idxv = sel[pl.ds(16 * k, 16)]
                rows = qoff + (iota16 + 16 * k) * 8
                for c, tab in enumerate((px, py, pz, f0, f1, f2)):
                    vals = plsc.load_gather(tab, [idxv])
                    if c == 0:
                        vals = vals - cxv
                    elif c == 1:
                        vals = vals - cyv
                    elif c == 2:
                        vals = vals - czv
                    plsc.store_scatter(stf, [rows + c], vals)
        return 0

    def gbody(g, _):
        lax.fori_loop(0, _QG, functools.partial(qbody, g), 0)
        base = (b * NQ + q0 + g * _QG)
        pltpu.sync_copy(st1, out1.at[pl.ds(base * NS1 * 8, _QG * NS1 * 8)])
        pltpu.sync_copy(st2, out2.at[pl.ds(base * NS2 * 8, _QG * NS2 * 8)])
        return 0

    lax.fori_loop(0, 128 // _QG, gbody, 0)


def _mlp_body(g1_ref, g2_ref, w10, b10, w11, b11, w12, b12,
              w20, b20, w21, b21, w22, b22, out_ref):
    def mlp(g, w0, b0, w1, b1, w2, b2, ns):
        h = jnp.maximum(jnp.dot(g, w0[...],
                                preferred_element_type=jnp.float32)
                        + b0[...], 0.0)
        h = jnp.maximum(jnp.dot(h, w1[...],
                                preferred_element_type=jnp.float32)
                        + b1[...], 0.0)
        h = jnp.maximum(jnp.dot(h, w2[...],
                                preferred_element_type=jnp.float32)
                        + b2[...], 0.0)
        return jnp.max(h.reshape(128, ns, 128), axis=1)

    o1 = mlp(g1_ref[...], w10, b10, w11, b11, w12, b12, NS1)
    o2 = mlp(g2_ref[...], w20, b20, w21, b21, w22, b22, NS2)
    out_ref[...] = jnp.concatenate([o1, o2], axis=1)


def _mlp(g1, g2, ws):
    nblk = B * NQ // 128
    wspecs = [pl.BlockSpec(w.shape, lambda i: (0, 0)) for w in ws]
    return pl.pallas_call(
        _mlp_body,
        grid=(nblk,),
        in_specs=[
            pl.BlockSpec((128 * NS1, 8), lambda i: (i, 0)),
            pl.BlockSpec((128 * NS2, 8), lambda i: (i, 0)),
            *wspecs,
        ],
        out_specs=pl.BlockSpec((128, 256), lambda i: (i, 0)),
        out_shape=jax.ShapeDtypeStruct((B * NQ, 256), jnp.float32),
    )(g1, g2, *ws)


def kernel(pointcloud, w1_0, b1_0, w1_1, b1_1, w1_2, b1_2,
           w2_0, b2_0, w2_1, b2_1, w2_2, b2_2):
    x = pointcloud[:, :, 0]
    y = pointcloud[:, :, 1]
    z = pointcloud[:, :, 2]
    f0 = pointcloud[:, :, 3]
    f1 = pointcloud[:, :, 4]
    f2 = pointcloud[:, :, 5]

    nxc = _fps(x.reshape(B, 128, 128), y.reshape(B, 128, 128),
               z.reshape(B, 128, 128))

    g1, g2 = _make_sc_group()(x, y, z, f0, f1, f2, nxc.reshape(NQ * 3 * B))
    g1 = g1.reshape(B * NQ * NS1, 8)
    g2 = g2.reshape(B * NQ * NS2, 8)

    def pad8(w):
        return jnp.zeros((8, w.shape[1]), jnp.float32).at[:6].set(w)

    ws = [pad8(w1_0), b1_0.reshape(1, -1), w1_1, b1_1.reshape(1, -1),
          w1_2, b1_2.reshape(1, -1),
          pad8(w2_0), b2_0.reshape(1, -1), w2_1, b2_1.reshape(1, -1),
          w2_2, b2_2.reshape(1, -1)]
    out = _mlp(g1, g2, ws)
    return out.reshape(B, NQ, 256)
